# speedup vs baseline: 1.9859x; 1.9859x over previous
import functools

import jax
import jax.numpy as jnp
from jax import lax
from jax.experimental import pallas as pl
from jax.experimental.pallas import tpu as pltpu
from jax.experimental.pallas import tpu_sc as plsc

BATCH = 4
SEQ = 8192
DIM = 1024
TOTAL = BATCH * SEQ
NUM_CORES = 2
NUM_SUBCORES = 16
NW = NUM_CORES * NUM_SUBCORES
BPW = TOTAL // NW
K = 32
NCHUNK = BPW // K

_mesh = plsc.VectorSubcoreMesh(core_axis_name="c", subcore_axis_name="s")


@functools.partial(
    pl.kernel,
    mesh=_mesh,
    out_type=jax.ShapeDtypeStruct((TOTAL, DIM), jnp.float32),
    scratch_types=[
        pltpu.VMEM((BPW,), jnp.int32),
        pltpu.VMEM((2, K, DIM), jnp.float32),
        pltpu.SemaphoreType.DMA,
    ],
)
def _gather_sc(idx_hbm, table_hbm, out_hbm, idx_v, rows_v, gsem):
    wid = lax.axis_index("s") * NUM_CORES + lax.axis_index("c")
    base = wid * BPW
    pltpu.sync_copy(idx_hbm.at[pl.ds(base, BPW)], idx_v)

    def body(c, _):
        pltpu.async_copy(
            table_hbm.at[idx_v.at[pl.ds(c * K, K)]], rows_v.at[0], gsem
        ).wait()
        pltpu.sync_copy(rows_v.at[0], out_hbm.at[pl.ds(base + c * K, K)])
        return 0

    lax.fori_loop(0, NCHUNK, body, 0)


def kernel(positional_idx, embedding):
    idx_flat = positional_idx.reshape(-1).astype(jnp.int32)
    out = _gather_sc(idx_flat, embedding)
    return out.reshape(BATCH, SEQ, DIM)


# speedup vs baseline: 2.3046x; 1.1605x over previous
import functools

import jax
import jax.numpy as jnp
from jax import lax
from jax.experimental import pallas as pl
from jax.experimental.pallas import tpu as pltpu
from jax.experimental.pallas import tpu_sc as plsc

BATCH = 4
SEQ = 8192
DIM = 1024
TOTAL = BATCH * SEQ
NUM_CORES = 2
NUM_SUBCORES = 16
NW = NUM_CORES * NUM_SUBCORES
BPW = TOTAL // NW
K = 32
NCHUNK = BPW // K

_mesh = plsc.VectorSubcoreMesh(core_axis_name="c", subcore_axis_name="s")


@functools.partial(
    pl.kernel,
    mesh=_mesh,
    out_type=jax.ShapeDtypeStruct((TOTAL, DIM), jnp.float32),
    scratch_types=[
        pltpu.VMEM((BPW,), jnp.int32),
        pltpu.VMEM((2, K, DIM), jnp.float32),
        pltpu.SemaphoreType.DMA,
        pltpu.SemaphoreType.DMA,
    ],
)
def _gather_sc(idx_hbm, table_hbm, out_hbm, idx_v, rows_v, sem0, sem1):
    wid = lax.axis_index("s") * NUM_CORES + lax.axis_index("c")
    base = wid * BPW
    sems = (sem0, sem1)
    pltpu.sync_copy(idx_hbm.at[pl.ds(base, BPW)], idx_v)

    def g_start(c, buf):
        pltpu.async_copy(
            table_hbm.at[idx_v.at[pl.ds(c * K, K)]], rows_v.at[buf], sems[buf]
        )

    def g_wait(c, buf):
        pltpu.make_async_copy(
            table_hbm.at[idx_v.at[pl.ds(c * K, K)]], rows_v.at[buf], sems[buf]
        ).wait()

    g_start(0, 0)

    def body(i, _):
        c0 = i * 2
        g_wait(c0, 0)
        g_start(c0 + 1, 1)
        pltpu.sync_copy(rows_v.at[0], out_hbm.at[pl.ds(base + c0 * K, K)])
        c1 = c0 + 1
        g_wait(c1, 1)

        @pl.when(c1 + 1 < NCHUNK)
        def _():
            g_start(c1 + 1, 0)

        pltpu.sync_copy(rows_v.at[1], out_hbm.at[pl.ds(base + c1 * K, K)])
        return 0

    lax.fori_loop(0, NCHUNK // 2, body, 0)


def kernel(positional_idx, embedding):
    idx_flat = positional_idx.reshape(-1).astype(jnp.int32)
    out = _gather_sc(idx_flat, embedding)
    return out.reshape(BATCH, SEQ, DIM)


# speedup vs baseline: 2.3723x; 1.0294x over previous
import functools

import jax
import jax.numpy as jnp
from jax import lax
from jax.experimental import pallas as pl
from jax.experimental.pallas import tpu as pltpu
from jax.experimental.pallas import tpu_sc as plsc

BATCH = 4
SEQ = 8192
DIM = 1024
TOTAL = BATCH * SEQ
NUM_CORES = 2
NUM_SUBCORES = 16
NW = NUM_CORES * NUM_SUBCORES
BPW = TOTAL // NW
K = 16
NCHUNK = BPW // K
NB = 4

_mesh = plsc.VectorSubcoreMesh(core_axis_name="c", subcore_axis_name="s")


@functools.partial(
    pl.kernel,
    mesh=_mesh,
    out_type=jax.ShapeDtypeStruct((TOTAL, DIM), jnp.float32),
    scratch_types=[
        pltpu.VMEM((BPW,), jnp.int32),
        pltpu.VMEM((NB, K, DIM), jnp.float32),
        pltpu.SemaphoreType.DMA,
        pltpu.SemaphoreType.DMA,
        pltpu.SemaphoreType.DMA,
        pltpu.SemaphoreType.DMA,
        pltpu.SemaphoreType.DMA,
        pltpu.SemaphoreType.DMA,
        pltpu.SemaphoreType.DMA,
        pltpu.SemaphoreType.DMA,
    ],
)
def _gather_sc(idx_hbm, table_hbm, out_hbm, idx_v, rows_v, g0, g1, g2, g3,
               w0, w1, w2, w3):
    wid = lax.axis_index("s") * NUM_CORES + lax.axis_index("c")
    base = wid * BPW
    gsems = (g0, g1, g2, g3)
    wsems = (w0, w1, w2, w3)
    pltpu.sync_copy(idx_hbm.at[pl.ds(base, BPW)], idx_v)

    def g_copy(c, buf):
        return pltpu.make_async_copy(
            table_hbm.at[idx_v.at[pl.ds(c * K, K)]], rows_v.at[buf], gsems[buf]
        )

    def w_copy(c, buf):
        return pltpu.make_async_copy(
            rows_v.at[buf], out_hbm.at[pl.ds(base + c * K, K)], wsems[buf]
        )

    for b in range(NB - 1):
        g_copy(b, b).start()

    for c in range(NB):
        g_copy(c, c).wait()
        w_copy(c, c).start()
        nxt = c + NB - 1
        if nxt < NCHUNK:
            if c >= 1:
                w_copy(c - 1, (c - 1) % NB).wait()
            g_copy(nxt, nxt % NB).start()

    def body(i, _):
        for b in range(NB):
            c = i * NB + b
            g_copy(c, b).wait()
            w_copy(c, b).start()
            nxt = c + NB - 1
            pb = (b + NB - 1) % NB

            @pl.when(nxt < NCHUNK)
            def _():
                w_copy(c - 1, pb).wait()
                g_copy(nxt, pb).start()

        return 0

    lax.fori_loop(1, NCHUNK // NB, body, 0)
    for b in range(NB):
        w_copy(NCHUNK - NB + b, b).wait()


def kernel(positional_idx, embedding):
    idx_flat = positional_idx.reshape(-1).astype(jnp.int32)
    out = _gather_sc(idx_flat, embedding)
    return out.reshape(BATCH, SEQ, DIM)
